# Initial kernel scaffold; baseline (speedup 1.0000x reference)
import jax
import jax.numpy as jnp
from jax.experimental import pallas as pl


def kernel(x, router, w_up_gate, w_down):
    raise NotImplementedError("write your pallas kernel here")


# speedup vs baseline: 3.3606x; 3.3606x over previous
import functools

import jax
import jax.numpy as jnp
from jax import lax
from jax.experimental import pallas as pl
from jax.experimental.pallas import tpu as pltpu
from jax.experimental.pallas import tpu_sc as plsc

_N = 2048
_D = 1024
_E = 64
_I = 2048
_K = 2
_A = _N * _K
_T = 128
_NB = 96
_ND = _NB * _T

_SC_CORES = 2
_SC_SUBCORES = 16
_SC_WORKERS = _SC_CORES * _SC_SUBCORES
_SC_CHUNK = 64


def _router_logits(x_flat, router):
    blk = 256

    def body(x_ref, r_ref, o_ref):
        o_ref[...] = jnp.dot(x_ref[...], r_ref[...],
                             preferred_element_type=jnp.float32)

    return pl.pallas_call(
        body,
        grid=(_N // blk,),
        in_specs=[
            pl.BlockSpec((blk, _D), lambda i: (i, 0)),
            pl.BlockSpec((_D, _E), lambda i: (0, 0)),
        ],
        out_specs=pl.BlockSpec((blk, _E), lambda i: (i, 0)),
        out_shape=jax.ShapeDtypeStruct((_N, _E), jnp.float32),
    )(x_flat, router)


def _sc_gather_rows(table, idx):
    b = idx.shape[0]
    b_per_w = b // _SC_WORKERS
    n_chunks = b_per_w // _SC_CHUNK
    mesh = plsc.VectorSubcoreMesh(core_axis_name="c", subcore_axis_name="s")

    @functools.partial(
        pl.kernel,
        mesh=mesh,
        out_type=jax.ShapeDtypeStruct((b, _D), jnp.float32),
        scratch_types=[
            pltpu.VMEM((b_per_w,), jnp.int32),
            pltpu.VMEM((_SC_CHUNK, _D), jnp.float32),
            pltpu.SemaphoreType.DMA,
        ],
    )
    def k(table_hbm, idx_hbm, out_hbm, idx_v, rows_v, sem):
        wid = lax.axis_index("s") * _SC_CORES + lax.axis_index("c")
        base = wid * b_per_w
        pltpu.sync_copy(idx_hbm.at[pl.ds(base, b_per_w)], idx_v)
        for c in range(n_chunks):
            pltpu.async_copy(
                table_hbm.at[idx_v.at[pl.ds(c * _SC_CHUNK, _SC_CHUNK)]],
                rows_v, sem).wait()
            pltpu.sync_copy(rows_v,
                            out_hbm.at[pl.ds(base + c * _SC_CHUNK, _SC_CHUNK)])

    return k(table, idx)


def _grouped_mlp(e_block, valid, xd, w_pad, w_up_gate, w_down):
    def body(eb_ref, vb_ref, xd_ref, w_ref, wug_ref, wd_ref, o_ref):
        j = pl.program_id(0)

        @pl.when(vb_ref[j] > 0)
        def _():
            xb = xd_ref[0]
            ug = jnp.dot(xb, wug_ref[0], preferred_element_type=jnp.float32)
            gate = ug[:, :_I]
            up = ug[:, _I:]
            h = (gate * jax.nn.sigmoid(gate)) * up
            y = jnp.dot(h, wd_ref[0], preferred_element_type=jnp.float32)
            o_ref[0] = y * w_ref[0]

    grid_spec = pltpu.PrefetchScalarGridSpec(
        num_scalar_prefetch=2,
        grid=(_NB,),
        in_specs=[
            pl.BlockSpec((1, _T, _D), lambda j, eb, vb: (j, 0, 0)),
            pl.BlockSpec((1, _T, 1), lambda j, eb, vb: (j, 0, 0)),
            pl.BlockSpec((1, _D, 2 * _I), lambda j, eb, vb: (eb[j], 0, 0)),
            pl.BlockSpec((1, _I, _D), lambda j, eb, vb: (eb[j], 0, 0)),
        ],
        out_specs=pl.BlockSpec((1, _T, _D), lambda j, eb, vb: (j, 0, 0)),
    )
    return pl.pallas_call(
        body,
        grid_spec=grid_spec,
        out_shape=jax.ShapeDtypeStruct((_NB, _T, _D), jnp.float32),
    )(e_block, valid, xd, w_pad, w_up_gate, w_down)


def _pair_add(g):
    blk = 256

    def body(a_ref, b_ref, o_ref):
        o_ref[...] = a_ref[...] + b_ref[...]

    nblk = _N // blk
    return pl.pallas_call(
        body,
        grid=(nblk,),
        in_specs=[
            pl.BlockSpec((blk, _D), lambda i: (i, 0)),
            pl.BlockSpec((blk, _D), lambda i, _n=nblk: (i + _n, 0)),
        ],
        out_specs=pl.BlockSpec((blk, _D), lambda i: (i, 0)),
        out_shape=jax.ShapeDtypeStruct((_N, _D), jnp.float32),
    )(g, g)


def kernel(x, router, w_up_gate, w_down):
    b, s, d = x.shape
    x_flat = x.reshape(_N, _D)

    logits = _router_logits(x_flat, router)
    top_vals, top_idx = jax.lax.top_k(logits, _K)
    combine = jax.nn.softmax(top_vals, axis=-1).astype(jnp.float32)

    expert_flat = top_idx.reshape(-1).astype(jnp.int32)
    token_flat = jnp.arange(_A, dtype=jnp.int32) // _K
    w_flat = combine.reshape(-1)

    order = jnp.argsort(expert_flat)
    sorted_expert = expert_flat[order]
    counts = jnp.bincount(expert_flat, length=_E).astype(jnp.int32)
    offsets = jnp.cumsum(counts) - counts
    nb_e = (counts + _T - 1) // _T
    cum_nb = jnp.cumsum(nb_e)
    total_nb = cum_nb[-1]
    padded_off = (cum_nb - nb_e) * _T

    jarr = jnp.arange(_NB, dtype=jnp.int32)
    e_block = jnp.minimum(
        jnp.searchsorted(cum_nb, jarr, side="right"), _E - 1
    ).astype(jnp.int32)
    valid = (jarr < total_nb).astype(jnp.int32)

    p = jnp.arange(_A, dtype=jnp.int32)
    pp = padded_off[sorted_expert] + (p - offsets[sorted_expert])
    tok_pad = jnp.zeros((_ND,), jnp.int32).at[pp].set(token_flat[order])
    w_pad = jnp.zeros((_ND,), jnp.float32).at[pp].set(w_flat[order])
    pp_of_a = jnp.zeros((_A,), jnp.int32).at[order].set(pp)
    gidx = jnp.concatenate([pp_of_a[0::_K], pp_of_a[1::_K]])

    xd = _sc_gather_rows(x_flat, tok_pad)

    yd = _grouped_mlp(e_block, valid,
                      xd.reshape(_NB, _T, _D),
                      w_pad.reshape(_NB, _T, 1),
                      w_up_gate, w_down)

    g = _sc_gather_rows(yd.reshape(_ND, _D), gidx)
    out = _pair_add(g)
    return out.reshape(b, s, d)


# speedup vs baseline: 3.3630x; 1.0007x over previous
import functools

import jax
import jax.numpy as jnp
from jax import lax
from jax.experimental import pallas as pl
from jax.experimental.pallas import tpu as pltpu
from jax.experimental.pallas import tpu_sc as plsc

_N = 2048
_D = 1024
_E = 64
_I = 2048
_K = 2
_A = _N * _K
_T = 128
_NB = 96
_ND = _NB * _T

_SC_CORES = 2
_SC_SUBCORES = 16
_SC_WORKERS = _SC_CORES * _SC_SUBCORES
_SC_CHUNK = 64


def _router_logits(x_flat, router):
    blk = 256

    def body(x_ref, r_ref, o_ref):
        o_ref[...] = jnp.dot(x_ref[...], r_ref[...],
                             preferred_element_type=jnp.float32)

    return pl.pallas_call(
        body,
        grid=(_N // blk,),
        in_specs=[
            pl.BlockSpec((blk, _D), lambda i: (i, 0)),
            pl.BlockSpec((_D, _E), lambda i: (0, 0)),
        ],
        out_specs=pl.BlockSpec((blk, _E), lambda i: (i, 0)),
        out_shape=jax.ShapeDtypeStruct((_N, _E), jnp.float32),
    )(x_flat, router)


def _sc_gather_rows(table, idx):
    b = idx.shape[0]
    b_per_w = b // _SC_WORKERS
    n_chunks = b_per_w // _SC_CHUNK
    mesh = plsc.VectorSubcoreMesh(core_axis_name="c", subcore_axis_name="s")

    @functools.partial(
        pl.kernel,
        mesh=mesh,
        out_type=jax.ShapeDtypeStruct((b, _D), jnp.float32),
        scratch_types=[
            pltpu.VMEM((b_per_w,), jnp.int32),
            pltpu.VMEM((_SC_CHUNK, _D), jnp.float32),
            pltpu.SemaphoreType.DMA,
        ],
    )
    def k(table_hbm, idx_hbm, out_hbm, idx_v, rows_v, sem):
        wid = lax.axis_index("s") * _SC_CORES + lax.axis_index("c")
        base = wid * b_per_w
        pltpu.sync_copy(idx_hbm.at[pl.ds(base, b_per_w)], idx_v)
        for c in range(n_chunks):
            pltpu.async_copy(
                table_hbm.at[idx_v.at[pl.ds(c * _SC_CHUNK, _SC_CHUNK)]],
                rows_v, sem).wait()
            pltpu.sync_copy(rows_v,
                            out_hbm.at[pl.ds(base + c * _SC_CHUNK, _SC_CHUNK)])

    return k(table, idx)


def _grouped_mlp(e_block, valid, xd, w_pad, w_up_gate, w_down):
    def body(eb_ref, vb_ref, xd_ref, w_ref, wug_ref, wd_ref, o_ref):
        j = pl.program_id(0)

        @pl.when(vb_ref[j] > 0)
        def _():
            xb = xd_ref[0].astype(jnp.bfloat16)
            wug = wug_ref[0].astype(jnp.bfloat16)
            ug = jnp.dot(xb, wug, preferred_element_type=jnp.float32)
            gate = ug[:, :_I]
            up = ug[:, _I:]
            h = (gate * jax.nn.sigmoid(gate)) * up
            y = jnp.dot(h.astype(jnp.bfloat16),
                        wd_ref[0].astype(jnp.bfloat16),
                        preferred_element_type=jnp.float32)
            o_ref[0] = y * w_ref[0]

    grid_spec = pltpu.PrefetchScalarGridSpec(
        num_scalar_prefetch=2,
        grid=(_NB,),
        in_specs=[
            pl.BlockSpec((1, _T, _D), lambda j, eb, vb: (j, 0, 0)),
            pl.BlockSpec((1, _T, 1), lambda j, eb, vb: (j, 0, 0)),
            pl.BlockSpec((1, _D, 2 * _I), lambda j, eb, vb: (eb[j], 0, 0)),
            pl.BlockSpec((1, _I, _D), lambda j, eb, vb: (eb[j], 0, 0)),
        ],
        out_specs=pl.BlockSpec((1, _T, _D), lambda j, eb, vb: (j, 0, 0)),
    )
    return pl.pallas_call(
        body,
        grid_spec=grid_spec,
        out_shape=jax.ShapeDtypeStruct((_NB, _T, _D), jnp.float32),
    )(e_block, valid, xd, w_pad, w_up_gate, w_down)


def _pair_add(g):
    blk = 256

    def body(a_ref, b_ref, o_ref):
        o_ref[...] = a_ref[...] + b_ref[...]

    nblk = _N // blk
    return pl.pallas_call(
        body,
        grid=(nblk,),
        in_specs=[
            pl.BlockSpec((blk, _D), lambda i: (i, 0)),
            pl.BlockSpec((blk, _D), lambda i, _n=nblk: (i + _n, 0)),
        ],
        out_specs=pl.BlockSpec((blk, _D), lambda i: (i, 0)),
        out_shape=jax.ShapeDtypeStruct((_N, _D), jnp.float32),
    )(g, g)


def kernel(x, router, w_up_gate, w_down):
    b, s, d = x.shape
    x_flat = x.reshape(_N, _D)

    logits = _router_logits(x_flat, router)
    top_vals, top_idx = jax.lax.top_k(logits, _K)
    combine = jax.nn.softmax(top_vals, axis=-1).astype(jnp.float32)

    expert_flat = top_idx.reshape(-1).astype(jnp.int32)
    token_flat = jnp.arange(_A, dtype=jnp.int32) // _K
    w_flat = combine.reshape(-1)

    order = jnp.argsort(expert_flat)
    sorted_expert = expert_flat[order]
    counts = jnp.bincount(expert_flat, length=_E).astype(jnp.int32)
    offsets = jnp.cumsum(counts) - counts
    nb_e = (counts + _T - 1) // _T
    cum_nb = jnp.cumsum(nb_e)
    total_nb = cum_nb[-1]
    padded_off = (cum_nb - nb_e) * _T

    jarr = jnp.arange(_NB, dtype=jnp.int32)
    e_block = jnp.minimum(
        jnp.searchsorted(cum_nb, jarr, side="right"), _E - 1
    ).astype(jnp.int32)
    valid = (jarr < total_nb).astype(jnp.int32)

    p = jnp.arange(_A, dtype=jnp.int32)
    pp = padded_off[sorted_expert] + (p - offsets[sorted_expert])
    tok_pad = jnp.zeros((_ND,), jnp.int32).at[pp].set(token_flat[order])
    w_pad = jnp.zeros((_ND,), jnp.float32).at[pp].set(w_flat[order])
    pp_of_a = jnp.zeros((_A,), jnp.int32).at[order].set(pp)
    gidx = jnp.concatenate([pp_of_a[0::_K], pp_of_a[1::_K]])

    xd = _sc_gather_rows(x_flat, tok_pad)

    yd = _grouped_mlp(e_block, valid,
                      xd.reshape(_NB, _T, _D),
                      w_pad.reshape(_NB, _T, 1),
                      w_up_gate, w_down)

    g = _sc_gather_rows(yd.reshape(_ND, _D), gidx)
    out = _pair_add(g)
    return out.reshape(b, s, d)


# speedup vs baseline: 4.9525x; 1.4726x over previous
import functools

import jax
import jax.numpy as jnp
from jax import lax
from jax.experimental import pallas as pl
from jax.experimental.pallas import tpu as pltpu
from jax.experimental.pallas import tpu_sc as plsc

_N = 2048
_D = 1024
_E = 64
_I = 2048
_K = 2
_A = _N * _K
_T = 128
_NB = 96
_ND = _NB * _T

_SC_CORES = 2
_SC_SUBCORES = 16
_SC_WORKERS = _SC_CORES * _SC_SUBCORES
_SC_CHUNK = 64


def _router_logits(x_flat, router):
    blk = 256

    def body(x_ref, r_ref, o_ref):
        o_ref[...] = jnp.dot(x_ref[...], r_ref[...],
                             preferred_element_type=jnp.float32)

    return pl.pallas_call(
        body,
        grid=(_N // blk,),
        in_specs=[
            pl.BlockSpec((blk, _D), lambda i: (i, 0)),
            pl.BlockSpec((_D, _E), lambda i: (0, 0)),
        ],
        out_specs=pl.BlockSpec((blk, _E), lambda i: (i, 0)),
        out_shape=jax.ShapeDtypeStruct((_N, _E), jnp.float32),
    )(x_flat, router)


def _sc_gather_rows(table, idx):
    b = idx.shape[0]
    b_per_w = b // _SC_WORKERS
    n_chunks = b_per_w // _SC_CHUNK
    mesh = plsc.VectorSubcoreMesh(core_axis_name="c", subcore_axis_name="s")

    @functools.partial(
        pl.kernel,
        mesh=mesh,
        out_type=jax.ShapeDtypeStruct((b, _D), jnp.float32),
        scratch_types=[
            pltpu.VMEM((b_per_w,), jnp.int32),
            pltpu.VMEM((_SC_CHUNK, _D), jnp.float32),
            pltpu.SemaphoreType.DMA,
        ],
    )
    def k(table_hbm, idx_hbm, out_hbm, idx_v, rows_v, sem):
        wid = lax.axis_index("s") * _SC_CORES + lax.axis_index("c")
        base = wid * b_per_w
        pltpu.sync_copy(idx_hbm.at[pl.ds(base, b_per_w)], idx_v)
        for c in range(n_chunks):
            pltpu.async_copy(
                table_hbm.at[idx_v.at[pl.ds(c * _SC_CHUNK, _SC_CHUNK)]],
                rows_v, sem).wait()
            pltpu.sync_copy(rows_v,
                            out_hbm.at[pl.ds(base + c * _SC_CHUNK, _SC_CHUNK)])

    return k(table, idx)


def _grouped_mlp(e_block, valid, xd, w_pad, w_up_gate, w_down):
    def body(eb_ref, vb_ref, xd_ref, w_ref, wug_ref, wd_ref, o_ref):
        j = pl.program_id(0)

        @pl.when(vb_ref[j] > 0)
        def _():
            xb = xd_ref[0].astype(jnp.bfloat16)
            wug = wug_ref[0].astype(jnp.bfloat16)
            ug = jnp.dot(xb, wug, preferred_element_type=jnp.float32)
            gate = ug[:, :_I]
            up = ug[:, _I:]
            h = (gate * jax.nn.sigmoid(gate)) * up
            y = jnp.dot(h.astype(jnp.bfloat16),
                        wd_ref[0].astype(jnp.bfloat16),
                        preferred_element_type=jnp.float32)
            o_ref[0] = y * w_ref[0]

    grid_spec = pltpu.PrefetchScalarGridSpec(
        num_scalar_prefetch=2,
        grid=(_NB,),
        in_specs=[
            pl.BlockSpec((1, _T, _D), lambda j, eb, vb: (j, 0, 0)),
            pl.BlockSpec((1, _T, 1), lambda j, eb, vb: (j, 0, 0)),
            pl.BlockSpec((1, _D, 2 * _I), lambda j, eb, vb: (eb[j], 0, 0)),
            pl.BlockSpec((1, _I, _D), lambda j, eb, vb: (eb[j], 0, 0)),
        ],
        out_specs=pl.BlockSpec((1, _T, _D), lambda j, eb, vb: (j, 0, 0)),
    )
    return pl.pallas_call(
        body,
        grid_spec=grid_spec,
        out_shape=jax.ShapeDtypeStruct((_NB, _T, _D), jnp.float32),
    )(e_block, valid, xd, w_pad, w_up_gate, w_down)


def _pair_add(g):
    blk = 256

    def body(a_ref, b_ref, o_ref):
        o_ref[...] = a_ref[...] + b_ref[...]

    nblk = _N // blk
    return pl.pallas_call(
        body,
        grid=(nblk,),
        in_specs=[
            pl.BlockSpec((blk, _D), lambda i: (i, 0)),
            pl.BlockSpec((blk, _D), lambda i, _n=nblk: (i + _n, 0)),
        ],
        out_specs=pl.BlockSpec((blk, _D), lambda i: (i, 0)),
        out_shape=jax.ShapeDtypeStruct((_N, _D), jnp.float32),
    )(g, g)


def kernel(x, router, w_up_gate, w_down):
    b, s, d = x.shape
    x_flat = x.reshape(_N, _D)

    logits = _router_logits(x_flat, router)
    top_vals, top_idx = jax.lax.top_k(logits, _K)
    combine = jax.nn.softmax(top_vals, axis=-1).astype(jnp.float32)

    expert_flat = top_idx.reshape(-1).astype(jnp.int32)
    token_flat = jnp.arange(_A, dtype=jnp.int32) // _K
    w_flat = combine.reshape(-1)

    order = jnp.argsort(expert_flat)
    sorted_expert = expert_flat[order]
    counts = jnp.bincount(expert_flat, length=_E).astype(jnp.int32)
    offsets = jnp.cumsum(counts) - counts
    nb_e = (counts + _T - 1) // _T
    cum_nb = jnp.cumsum(nb_e)
    total_nb = cum_nb[-1]
    padded_off = (cum_nb - nb_e) * _T

    jarr = jnp.arange(_NB, dtype=jnp.int32)
    e_block = jnp.minimum(
        jnp.searchsorted(cum_nb, jarr, side="right"), _E - 1
    ).astype(jnp.int32)
    valid = (jarr < total_nb).astype(jnp.int32)

    p = jnp.arange(_A, dtype=jnp.int32)
    pp = padded_off[sorted_expert] + (p - offsets[sorted_expert])
    pad_base = jnp.arange(_ND, dtype=jnp.int32) % _N
    tok_pad = pad_base.at[pp].set(token_flat[order])
    w_pad = jnp.zeros((_ND,), jnp.float32).at[pp].set(w_flat[order])
    pp_of_a = jnp.zeros((_A,), jnp.int32).at[order].set(pp)
    gidx = jnp.concatenate([pp_of_a[0::_K], pp_of_a[1::_K]])

    xd = _sc_gather_rows(x_flat, tok_pad)

    yd = _grouped_mlp(e_block, valid,
                      xd.reshape(_NB, _T, _D),
                      w_pad.reshape(_NB, _T, 1),
                      w_up_gate, w_down)

    g = _sc_gather_rows(yd.reshape(_ND, _D), gidx)
    out = _pair_add(g)
    return out.reshape(b, s, d)
